# Initial kernel scaffold; baseline (speedup 1.0000x reference)
import jax
import jax.numpy as jnp
from jax.experimental import pallas as pl


def kernel(good_tokens, bad_tokens, t, perm_noise, W_in, W_t, W_q, W_k, W_v, W_o, W_head, W_d1, W_d2, w_cls):
    raise NotImplementedError("write your pallas kernel here")


# speedup vs baseline: 1.0881x; 1.0881x over previous
import math

import jax
import jax.numpy as jnp
from jax.experimental import pallas as pl
from jax.experimental.pallas import tpu as pltpu

TIMESTEPS = 1000
MAX_OUTLIERS = 128
N = 256
B = 8
N_GOOD = 128
D = 256
DM = 256

_HI = jax.lax.Precision.HIGHEST


def _ratio_table():
    epsilon = 0.008
    steps = jnp.linspace(0.0, TIMESTEPS, TIMESTEPS + 1, dtype=jnp.float32)
    f_t = jnp.cos((steps / TIMESTEPS + epsilon) / (1.0 + epsilon) * math.pi * 0.5) ** 2
    betas = jnp.clip(1.0 - f_t[1:] / f_t[:TIMESTEPS], 0.0, 0.999)
    alphas_cumprod = jnp.cumprod(1.0 - betas)
    tab = jnp.sqrt(1.0 - alphas_cumprod)
    tab = jnp.concatenate([tab, jnp.zeros((24,), jnp.float32)])
    return tab.reshape(8, 128)


def _kern(t_sref, g_ref, b_ref, pnr_ref, pnc_ref, tab_ref,
          Win, Wt, Wq, Wk, Wv, Wo, Wh, Wd1, Wd2, wc, out_ref):
    b = pl.program_id(0)
    t = t_sref[b]
    tf = t.astype(jnp.float32)

    r8 = jax.lax.broadcasted_iota(jnp.int32, (8, 128), 0)
    c8 = jax.lax.broadcasted_iota(jnp.int32, (8, 128), 1)
    ratio = jnp.sum(jnp.where(r8 * 128 + c8 == t, tab_ref[...], 0.0))
    outlier = jnp.floor(MAX_OUTLIERS * ratio).astype(jnp.int32)
    datanum = N_GOOD + outlier

    io_i = jax.lax.broadcasted_iota(jnp.int32, (N, N), 0)
    io_j = jax.lax.broadcasted_iota(jnp.int32, (N, N), 1)

    x = jnp.concatenate([g_ref[0], b_ref[0]], axis=0)
    x = jnp.where(io_i < datanum, x, 1.0)

    h1 = jax.nn.relu(jnp.dot(x, Wd1[...], precision=_HI))
    h2 = jax.nn.relu(jnp.dot(h1, Wd2[...], precision=_HI))
    pooled = jnp.concatenate(
        [jnp.max(h2, axis=0, keepdims=True), jnp.mean(h2, axis=0, keepdims=True)],
        axis=1)
    pt_row = jax.nn.sigmoid(jnp.dot(pooled, wc[...], precision=_HI))

    xi = jnp.where(x >= 0.0, jnp.floor(x), jnp.ceil(x))
    lane = jax.lax.broadcasted_iota(jnp.int32, (1, 128), 1).astype(jnp.float32)
    freqs = jnp.exp(-(math.log(10000.0) / 128.0) * lane)
    ang = tf * freqs
    temb_raw = jnp.concatenate([jnp.sin(ang), jnp.cos(ang)], axis=1)
    temb = jnp.dot(temb_raw, Wt[...], precision=_HI)
    feats = jnp.dot(xi, Win[...], precision=_HI) + temb
    q = jnp.dot(feats, Wq[...], precision=_HI)
    k = jnp.dot(feats, Wk[...], precision=_HI)
    v = jnp.dot(feats, Wv[...], precision=_HI)
    scores = jax.lax.dot_general(q, k, (((1,), (1,)), ((), ())),
                                 precision=_HI) * (1.0 / 16.0)
    m = jnp.max(scores, axis=1, keepdims=True)
    e = jnp.exp(scores - m)
    attn = e / jnp.sum(e, axis=1, keepdims=True)
    av = jnp.dot(attn, v, precision=_HI)
    out = feats + jnp.dot(av, Wo[...], precision=_HI)
    logits = jnp.dot(jax.nn.relu(out), Wh[...], precision=_HI)

    padr = jnp.where(io_j >= datanum, 1e6, 0.0)
    padc = jnp.where(io_i >= datanum, 1e6, 0.0)
    keys_j = pnr_ref[0] + padr
    keys_i = pnc_ref[0][:, 0:1] + padc
    before = (keys_j < keys_i) | ((keys_j == keys_i) & (io_j < io_i))
    rank = jnp.sum(before.astype(jnp.float32), axis=1, keepdims=True)
    io_jf = io_j.astype(jnp.float32)
    Q = (rank == io_jf).astype(jnp.float32)

    icol = io_i[:, 0:1]
    lab = jnp.where(icol < N_GOOD, 0.0, jnp.where(icol < datanum, 1.0, -1.0))
    ch = jax.lax.broadcasted_iota(jnp.int32, (N, 128), 1)
    M = logits + jnp.where(ch == 2, lab, 0.0)

    final = jax.lax.dot_general(M, Q, (((0,), (0,)), ((), ())),
                                precision=_HI)

    si = jax.lax.broadcasted_iota(jnp.int32, (128, 256), 0)
    li = jax.lax.broadcasted_iota(jnp.int32, (128, 256), 1)
    pt_full = jnp.concatenate(
        [jnp.broadcast_to(pt_row, (128, 128)), jnp.zeros((128, 128), jnp.float32)],
        axis=1)
    final = jnp.where((si == 3) & (li == 0), pt_full, final)
    final = jnp.where((si == 4) & (li == 0), tf * (1.0 / TIMESTEPS), final)
    out_ref[0] = final


def kernel(good_tokens, bad_tokens, t, perm_noise,
           W_in, W_t, W_q, W_k, W_v, W_o, W_head, W_d1, W_d2, w_cls):
    tab = _ratio_table()
    pn_row = perm_noise.reshape(B, 1, N)
    pn_col = jnp.broadcast_to(perm_noise[:, :, None], (B, N, 128))
    Wh128 = jnp.pad(W_head, ((0, 0), (0, 126)))
    wc128 = jnp.pad(w_cls, ((0, 0), (0, 127)))

    full2d = lambda s: pl.BlockSpec(s, lambda i, *_: (0, 0))
    per_b = lambda s: pl.BlockSpec(s, lambda i, *_: (i, 0, 0))

    grid_spec = pltpu.PrefetchScalarGridSpec(
        num_scalar_prefetch=1,
        grid=(B,),
        in_specs=[
            per_b((1, N_GOOD, D)),
            per_b((1, N - N_GOOD, D)),
            per_b((1, 1, N)),
            per_b((1, N, 128)),
            full2d((8, 128)),
            full2d((D, DM)),
            full2d((DM, DM)),
            full2d((DM, DM)),
            full2d((DM, DM)),
            full2d((DM, DM)),
            full2d((DM, DM)),
            full2d((DM, 128)),
            full2d((D, 128)),
            full2d((128, 128)),
            full2d((256, 128)),
        ],
        out_specs=per_b((1, 128, N)),
    )

    y = pl.pallas_call(
        _kern,
        grid_spec=grid_spec,
        out_shape=jax.ShapeDtypeStruct((B, 128, N), jnp.float32),
    )(t, good_tokens, bad_tokens, pn_row, pn_col, tab,
      W_in, W_t, W_q, W_k, W_v, W_o, Wh128, W_d1, W_d2, wc128)

    pred_label = y[:, :2, :]
    gt_label = y[:, 2, :].astype(jnp.int32)
    pred_t = y[:, 3, 0]
    gt_t = y[:, 4, 0]
    return pred_label, gt_label, pred_t, gt_t


# speedup vs baseline: 1.9340x; 1.7773x over previous
import math

import jax
import jax.numpy as jnp
from jax.experimental import pallas as pl
from jax.experimental.pallas import tpu as pltpu

TIMESTEPS = 1000
MAX_OUTLIERS = 128
N = 256
B = 8
N_GOOD = 128
D = 256
DM = 256

_HI = jax.lax.Precision.DEFAULT


def _ratio_table():
    epsilon = 0.008
    steps = jnp.linspace(0.0, TIMESTEPS, TIMESTEPS + 1, dtype=jnp.float32)
    f_t = jnp.cos((steps / TIMESTEPS + epsilon) / (1.0 + epsilon) * math.pi * 0.5) ** 2
    betas = jnp.clip(1.0 - f_t[1:] / f_t[:TIMESTEPS], 0.0, 0.999)
    alphas_cumprod = jnp.cumprod(1.0 - betas)
    tab = jnp.sqrt(1.0 - alphas_cumprod)
    tab = jnp.concatenate([tab, jnp.zeros((24,), jnp.float32)])
    return tab.reshape(8, 128)


def _kern(t_sref, g_ref, b_ref, pnr_ref, pnc_ref, tab_ref,
          Win, Wt, Wq, Wk, Wv, Wo, Wh, Wd1, Wd2, wc, out_ref):
    b = pl.program_id(0)
    t = t_sref[b]
    tf = t.astype(jnp.float32)

    r8 = jax.lax.broadcasted_iota(jnp.int32, (8, 128), 0)
    c8 = jax.lax.broadcasted_iota(jnp.int32, (8, 128), 1)
    ratio = jnp.sum(jnp.where(r8 * 128 + c8 == t, tab_ref[...], 0.0))
    outlier = jnp.floor(MAX_OUTLIERS * ratio).astype(jnp.int32)
    datanum = N_GOOD + outlier

    io_i = jax.lax.broadcasted_iota(jnp.int32, (N, N), 0)
    io_j = jax.lax.broadcasted_iota(jnp.int32, (N, N), 1)

    x = jnp.concatenate([g_ref[0], b_ref[0]], axis=0)
    x = jnp.where(io_i < datanum, x, 1.0)

    h1 = jax.nn.relu(jnp.dot(x, Wd1[...], precision=_HI))
    h2 = jax.nn.relu(jnp.dot(h1, Wd2[...], precision=_HI))
    pooled = jnp.concatenate(
        [jnp.max(h2, axis=0, keepdims=True), jnp.mean(h2, axis=0, keepdims=True)],
        axis=1)
    pt_row = jax.nn.sigmoid(jnp.dot(pooled, wc[...], precision=_HI))

    xi = jnp.where(x >= 0.0, jnp.floor(x), jnp.ceil(x))
    lane = jax.lax.broadcasted_iota(jnp.int32, (1, 128), 1).astype(jnp.float32)
    freqs = jnp.exp(-(math.log(10000.0) / 128.0) * lane)
    ang = tf * freqs
    temb_raw = jnp.concatenate([jnp.sin(ang), jnp.cos(ang)], axis=1)
    temb = jnp.dot(temb_raw, Wt[...], precision=_HI)
    feats = jnp.dot(xi, Win[...], precision=_HI) + temb
    q = jnp.dot(feats, Wq[...], precision=_HI)
    k = jnp.dot(feats, Wk[...], precision=_HI)
    v = jnp.dot(feats, Wv[...], precision=_HI)
    scores = jax.lax.dot_general(q, k, (((1,), (1,)), ((), ())),
                                 precision=_HI) * (1.0 / 16.0)
    m = jnp.max(scores, axis=1, keepdims=True)
    e = jnp.exp(scores - m)
    attn = e / jnp.sum(e, axis=1, keepdims=True)
    av = jnp.dot(attn, v, precision=_HI)
    out = feats + jnp.dot(av, Wo[...], precision=_HI)
    logits = jnp.dot(jax.nn.relu(out), Wh[...], precision=_HI)

    padr = jnp.where(io_j >= datanum, 1e6, 0.0)
    padc = jnp.where(io_i >= datanum, 1e6, 0.0)
    keys_j = pnr_ref[0] + padr
    keys_i = pnc_ref[0][:, 0:1] + padc
    before = (keys_j < keys_i) | ((keys_j == keys_i) & (io_j < io_i))
    rank = jnp.sum(before.astype(jnp.float32), axis=1, keepdims=True)
    io_jf = io_j.astype(jnp.float32)
    Q = (rank == io_jf).astype(jnp.float32)

    icol = io_i[:, 0:1]
    lab = jnp.where(icol < N_GOOD, 0.0, jnp.where(icol < datanum, 1.0, -1.0))
    ch = jax.lax.broadcasted_iota(jnp.int32, (N, 128), 1)
    M = logits + jnp.where(ch == 2, lab, 0.0)

    final = jax.lax.dot_general(M, Q, (((0,), (0,)), ((), ())),
                                precision=_HI)

    si = jax.lax.broadcasted_iota(jnp.int32, (128, 256), 0)
    li = jax.lax.broadcasted_iota(jnp.int32, (128, 256), 1)
    pt_full = jnp.concatenate(
        [jnp.broadcast_to(pt_row, (128, 128)), jnp.zeros((128, 128), jnp.float32)],
        axis=1)
    final = jnp.where((si == 3) & (li == 0), pt_full, final)
    final = jnp.where((si == 4) & (li == 0), tf * (1.0 / TIMESTEPS), final)
    out_ref[0] = final


def kernel(good_tokens, bad_tokens, t, perm_noise,
           W_in, W_t, W_q, W_k, W_v, W_o, W_head, W_d1, W_d2, w_cls):
    tab = _ratio_table()
    pn_row = perm_noise.reshape(B, 1, N)
    pn_col = jnp.broadcast_to(perm_noise[:, :, None], (B, N, 128))
    Wh128 = jnp.pad(W_head, ((0, 0), (0, 126)))
    wc128 = jnp.pad(w_cls, ((0, 0), (0, 127)))

    full2d = lambda s: pl.BlockSpec(s, lambda i, *_: (0, 0))
    per_b = lambda s: pl.BlockSpec(s, lambda i, *_: (i, 0, 0))

    grid_spec = pltpu.PrefetchScalarGridSpec(
        num_scalar_prefetch=1,
        grid=(B,),
        in_specs=[
            per_b((1, N_GOOD, D)),
            per_b((1, N - N_GOOD, D)),
            per_b((1, 1, N)),
            per_b((1, N, 128)),
            full2d((8, 128)),
            full2d((D, DM)),
            full2d((DM, DM)),
            full2d((DM, DM)),
            full2d((DM, DM)),
            full2d((DM, DM)),
            full2d((DM, DM)),
            full2d((DM, 128)),
            full2d((D, 128)),
            full2d((128, 128)),
            full2d((256, 128)),
        ],
        out_specs=per_b((1, 128, N)),
    )

    y = pl.pallas_call(
        _kern,
        grid_spec=grid_spec,
        out_shape=jax.ShapeDtypeStruct((B, 128, N), jnp.float32),
    )(t, good_tokens, bad_tokens, pn_row, pn_col, tab,
      W_in, W_t, W_q, W_k, W_v, W_o, Wh128, W_d1, W_d2, wc128)

    pred_label = y[:, :2, :]
    gt_label = y[:, 2, :].astype(jnp.int32)
    pred_t = y[:, 3, 0]
    gt_t = y[:, 4, 0]
    return pred_label, gt_label, pred_t, gt_t


# speedup vs baseline: 2.2209x; 1.1484x over previous
import math

import jax
import jax.numpy as jnp
from jax.experimental import pallas as pl
from jax.experimental.pallas import tpu as pltpu

TIMESTEPS = 1000
MAX_OUTLIERS = 128
N = 256
B = 8
SPS = 2
N_GOOD = 128
D = 256
DM = 256


def _ratio_table():
    epsilon = 0.008
    steps = jnp.linspace(0.0, TIMESTEPS, TIMESTEPS + 1, dtype=jnp.float32)
    f_t = jnp.cos((steps / TIMESTEPS + epsilon) / (1.0 + epsilon) * math.pi * 0.5) ** 2
    betas = jnp.clip(1.0 - f_t[1:] / f_t[:TIMESTEPS], 0.0, 0.999)
    alphas_cumprod = jnp.cumprod(1.0 - betas)
    tab = jnp.sqrt(1.0 - alphas_cumprod)
    tab = jnp.concatenate([tab, jnp.zeros((24,), jnp.float32)])
    return tab.reshape(8, 128)


def _one_sample(t, g, bd, pn_row, tab):
    tf = t.astype(jnp.float32)

    r8 = jax.lax.broadcasted_iota(jnp.int32, (8, 128), 0)
    c8 = jax.lax.broadcasted_iota(jnp.int32, (8, 128), 1)
    ratio = jnp.sum(jnp.where(r8 * 128 + c8 == t, tab, 0.0))
    outlier = jnp.floor(MAX_OUTLIERS * ratio).astype(jnp.int32)
    datanum = N_GOOD + outlier

    io_i = jax.lax.broadcasted_iota(jnp.int32, (N, N), 0)
    io_j = jax.lax.broadcasted_iota(jnp.int32, (N, N), 1)

    x = jnp.concatenate([g, bd], axis=0)
    x = jnp.where(io_i < datanum, x, 1.0)

    h1 = jax.nn.relu(jnp.dot(x, _one_sample.Wd1))
    h2 = jax.nn.relu(jnp.dot(h1, _one_sample.Wd2))
    pooled = jnp.concatenate(
        [jnp.max(h2, axis=0, keepdims=True), jnp.mean(h2, axis=0, keepdims=True)],
        axis=1)
    pt_row = jax.nn.sigmoid(jnp.dot(pooled, _one_sample.wc))

    xi = jnp.where(x >= 0.0, jnp.floor(x), jnp.ceil(x))
    lane = jax.lax.broadcasted_iota(jnp.int32, (1, 128), 1).astype(jnp.float32)
    freqs = jnp.exp(-(math.log(10000.0) / 128.0) * lane)
    ang = tf * freqs
    temb_raw = jnp.concatenate([jnp.sin(ang), jnp.cos(ang)], axis=1)
    temb = jnp.dot(temb_raw, _one_sample.Wt)
    feats = jnp.dot(xi, _one_sample.Win) + temb
    q = jnp.dot(feats, _one_sample.Wq)
    k = jnp.dot(feats, _one_sample.Wk)
    v = jnp.dot(feats, _one_sample.Wv)
    scores = jax.lax.dot_general(q, k, (((1,), (1,)), ((), ()))) * (1.0 / 16.0)
    m = jnp.max(scores, axis=1, keepdims=True)
    e = jnp.exp(scores - m)
    attn = e / jnp.sum(e, axis=1, keepdims=True)
    av = jnp.dot(attn, v)
    out = feats + jnp.dot(av, _one_sample.Wo)
    logits = jnp.dot(jax.nn.relu(out), _one_sample.Wh)

    keys_j = jnp.broadcast_to(pn_row, (N, N)) + jnp.where(io_j >= datanum, 1e6, 0.0)
    keys_i = keys_j.T
    before = (keys_j < keys_i) | ((keys_j == keys_i) & (io_j < io_i))
    rank = jnp.sum(before.astype(jnp.float32), axis=1, keepdims=True)
    Q = (rank == io_j.astype(jnp.float32)).astype(jnp.float32)

    icol = io_i[:, 0:1]
    lab = jnp.where(icol < N_GOOD, 0.0, jnp.where(icol < datanum, 1.0, -1.0))
    ch = jax.lax.broadcasted_iota(jnp.int32, (N, 128), 1)
    M = logits + jnp.where(ch == 2, lab, 0.0)

    final = jax.lax.dot_general(M, Q, (((0,), (0,)), ((), ())))[0:8, :]

    si = jax.lax.broadcasted_iota(jnp.int32, (8, 256), 0)
    li = jax.lax.broadcasted_iota(jnp.int32, (8, 256), 1)
    pt_full = jnp.concatenate(
        [jnp.broadcast_to(pt_row, (8, 128)), jnp.zeros((8, 128), jnp.float32)],
        axis=1)
    final = jnp.where((si == 3) & (li == 0), pt_full, final)
    final = jnp.where((si == 4) & (li == 0), tf * (1.0 / TIMESTEPS), final)
    return final


def _kern(t_sref, g_ref, b_ref, pn_ref, tab_ref,
          Win, Wt, Wq, Wk, Wv, Wo, Wh, Wd1, Wd2, wc, out_ref):
    step = pl.program_id(0)
    _one_sample.Win = Win[...]
    _one_sample.Wt = Wt[...]
    _one_sample.Wq = Wq[...]
    _one_sample.Wk = Wk[...]
    _one_sample.Wv = Wv[...]
    _one_sample.Wo = Wo[...]
    _one_sample.Wh = Wh[...]
    _one_sample.Wd1 = Wd1[...]
    _one_sample.Wd2 = Wd2[...]
    _one_sample.wc = wc[...]
    tab = tab_ref[...]
    for s in range(SPS):
        t = t_sref[step * SPS + s]
        out_ref[s] = _one_sample(t, g_ref[s], b_ref[s], pn_ref[s], tab)


def kernel(good_tokens, bad_tokens, t, perm_noise,
           W_in, W_t, W_q, W_k, W_v, W_o, W_head, W_d1, W_d2, w_cls):
    tab = _ratio_table()
    pn_row = perm_noise.reshape(B, 1, N)
    Wh128 = jnp.pad(W_head, ((0, 0), (0, 126)))
    wc128 = jnp.pad(w_cls, ((0, 0), (0, 127)))

    full2d = lambda s: pl.BlockSpec(s, lambda i, *_: (0, 0))
    per_b = lambda s: pl.BlockSpec(s, lambda i, *_: (i, 0, 0))

    grid_spec = pltpu.PrefetchScalarGridSpec(
        num_scalar_prefetch=1,
        grid=(B // SPS,),
        in_specs=[
            per_b((SPS, N_GOOD, D)),
            per_b((SPS, N - N_GOOD, D)),
            per_b((SPS, 1, N)),
            full2d((8, 128)),
            full2d((D, DM)),
            full2d((DM, DM)),
            full2d((DM, DM)),
            full2d((DM, DM)),
            full2d((DM, DM)),
            full2d((DM, DM)),
            full2d((DM, 128)),
            full2d((D, 128)),
            full2d((128, 128)),
            full2d((256, 128)),
        ],
        out_specs=per_b((SPS, 8, N)),
    )

    y = pl.pallas_call(
        _kern,
        grid_spec=grid_spec,
        out_shape=jax.ShapeDtypeStruct((B, 8, N), jnp.float32),
    )(t, good_tokens, bad_tokens, pn_row, tab,
      W_in, W_t, W_q, W_k, W_v, W_o, Wh128, W_d1, W_d2, wc128)

    pred_label = y[:, :2, :]
    gt_label = y[:, 2, :].astype(jnp.int32)
    pred_t = y[:, 3, 0]
    gt_t = y[:, 4, 0]
    return pred_label, gt_label, pred_t, gt_t


# speedup vs baseline: 2.6113x; 1.1758x over previous
import math

import jax
import jax.numpy as jnp
from jax.experimental import pallas as pl
from jax.experimental.pallas import tpu as pltpu

TIMESTEPS = 1000
MAX_OUTLIERS = 128
N = 256
B = 8
SPS = 2
N_GOOD = 128
D = 256
DM = 256


def _ratio_table():
    epsilon = 0.008
    steps = jnp.linspace(0.0, TIMESTEPS, TIMESTEPS + 1, dtype=jnp.float32)
    f_t = jnp.cos((steps / TIMESTEPS + epsilon) / (1.0 + epsilon) * math.pi * 0.5) ** 2
    betas = jnp.clip(1.0 - f_t[1:] / f_t[:TIMESTEPS], 0.0, 0.999)
    alphas_cumprod = jnp.cumprod(1.0 - betas)
    tab = jnp.sqrt(1.0 - alphas_cumprod)
    tab = jnp.concatenate([tab, jnp.zeros((24,), jnp.float32)])
    return tab.reshape(8, 128)


def _one_sample(W, t, g, bd, pn_row, tab, temb):
    tf = t.astype(jnp.float32)

    r8 = jax.lax.broadcasted_iota(jnp.int32, (8, 128), 0)
    c8 = jax.lax.broadcasted_iota(jnp.int32, (8, 128), 1)
    ratio = jnp.sum(jnp.where(r8 * 128 + c8 == t, tab, 0.0))
    outlier = jnp.floor(MAX_OUTLIERS * ratio).astype(jnp.int32)
    datanum = N_GOOD + outlier

    io_i = jax.lax.broadcasted_iota(jnp.int32, (N, N), 0)
    io_j = jax.lax.broadcasted_iota(jnp.int32, (N, N), 1)

    x = jnp.concatenate([g, bd], axis=0)
    x = jnp.where(io_i < datanum, x, 1.0)

    h1 = jax.nn.relu(jnp.dot(x, W['Wd1']))
    h2 = jax.nn.relu(jnp.dot(h1, W['Wd2']))
    pooled = jnp.concatenate(
        [jnp.max(h2, axis=0, keepdims=True), jnp.mean(h2, axis=0, keepdims=True)],
        axis=1)
    pt_row = jax.nn.sigmoid(jnp.dot(pooled, W['wc']))

    xi = jnp.where(x >= 0.0, jnp.floor(x), jnp.ceil(x))
    feats = jnp.dot(xi, W['Win']) + temb
    q = jnp.dot(feats, W['Wq'])
    k = jnp.dot(feats, W['Wk'])
    v = jnp.dot(feats, W['Wv'])
    scores = jax.lax.dot_general(q, k, (((1,), (1,)), ((), ()))) * (1.0 / 16.0)
    m = jnp.max(scores, axis=1, keepdims=True)
    e = jnp.exp(scores - m)
    attn = e / jnp.sum(e, axis=1, keepdims=True)
    av = jnp.dot(attn, v)
    out = feats + jnp.dot(av, W['Wo'])
    logits = jnp.dot(jax.nn.relu(out), W['Wh'])

    keys_j = jnp.broadcast_to(pn_row, (N, N)) + jnp.where(io_j >= datanum, 1e6, 0.0)
    keys_i = keys_j.T
    before = (keys_j < keys_i) | ((keys_j == keys_i) & (io_j < io_i))
    rank = jnp.sum(before.astype(jnp.float32), axis=1, keepdims=True)
    Q = (rank == io_j.astype(jnp.float32)).astype(jnp.float32)

    icol = io_i[:, 0:1]
    lab = jnp.where(icol < N_GOOD, 0.0, jnp.where(icol < datanum, 1.0, -1.0))
    ch = jax.lax.broadcasted_iota(jnp.int32, (N, 128), 1)
    M = logits + jnp.where(ch == 2, lab, 0.0)

    final = jax.lax.dot_general(M, Q, (((0,), (0,)), ((), ())))

    gt = jnp.round(final[2:3, :]).astype(jnp.int32)
    li = jax.lax.broadcasted_iota(jnp.int32, (1, 256), 1)
    pt_ext = jnp.concatenate([pt_row, jnp.zeros((1, 128), jnp.float32)], axis=1)
    misc = jnp.where(li == 0, pt_ext,
                     jnp.where(li == 1, tf * (1.0 / TIMESTEPS), 0.0))
    return final[0:2, :], gt, misc


def _kern(t_sref, g_ref, b_ref, pn_ref, tab_ref,
          Win, Wt, Wq, Wk, Wv, Wo, Wh, Wd1, Wd2, wc,
          pl_ref, gt_ref, misc_ref):
    step = pl.program_id(0)
    W = dict(Win=Win[...], Wq=Wq[...], Wk=Wk[...], Wv=Wv[...], Wo=Wo[...],
             Wh=Wh[...], Wd1=Wd1[...], Wd2=Wd2[...], wc=wc[...])
    tab = tab_ref[...]

    lane = jax.lax.broadcasted_iota(jnp.int32, (SPS, 128), 1).astype(jnp.float32)
    tf_col = jnp.stack([t_sref[step * SPS + s].astype(jnp.float32)
                        for s in range(SPS)])[:, None]
    ang = tf_col * jnp.exp(-(math.log(10000.0) / 128.0) * lane)
    temb_all = jnp.dot(jnp.concatenate([jnp.sin(ang), jnp.cos(ang)], axis=1),
                       Wt[...])

    for s in range(SPS):
        t = t_sref[step * SPS + s]
        logits2, gt, misc = _one_sample(W, t, g_ref[s], b_ref[s], pn_ref[s],
                                        tab, temb_all[s:s + 1, :])
        pl_ref[s] = logits2
        gt_ref[s] = gt
        misc_ref[s] = misc


def kernel(good_tokens, bad_tokens, t, perm_noise,
           W_in, W_t, W_q, W_k, W_v, W_o, W_head, W_d1, W_d2, w_cls):
    tab = _ratio_table()
    pn_row = perm_noise.reshape(B, 1, N)
    Wh128 = jnp.pad(W_head, ((0, 0), (0, 126)))
    wc128 = jnp.pad(w_cls, ((0, 0), (0, 127)))

    full2d = lambda s: pl.BlockSpec(s, lambda i, *_: (0, 0))
    per_b = lambda s: pl.BlockSpec(s, lambda i, *_: (i, 0, 0))

    grid_spec = pltpu.PrefetchScalarGridSpec(
        num_scalar_prefetch=1,
        grid=(B // SPS,),
        in_specs=[
            per_b((SPS, N_GOOD, D)),
            per_b((SPS, N - N_GOOD, D)),
            per_b((SPS, 1, N)),
            full2d((8, 128)),
            full2d((D, DM)),
            full2d((DM, DM)),
            full2d((DM, DM)),
            full2d((DM, DM)),
            full2d((DM, DM)),
            full2d((DM, DM)),
            full2d((DM, 128)),
            full2d((D, 128)),
            full2d((128, 128)),
            full2d((256, 128)),
        ],
        out_specs=[
            per_b((SPS, 2, N)),
            per_b((SPS, 1, N)),
            per_b((SPS, 1, N)),
        ],
    )

    pred_label, gt3, misc = pl.pallas_call(
        _kern,
        grid_spec=grid_spec,
        out_shape=[
            jax.ShapeDtypeStruct((B, 2, N), jnp.float32),
            jax.ShapeDtypeStruct((B, 1, N), jnp.int32),
            jax.ShapeDtypeStruct((B, 1, N), jnp.float32),
        ],
    )(t, good_tokens, bad_tokens, pn_row, tab,
      W_in, W_t, W_q, W_k, W_v, W_o, Wh128, W_d1, W_d2, wc128)

    gt_label = gt3.reshape(B, N)
    pred_t = misc[:, 0, 0]
    gt_t = misc[:, 0, 1]
    return pred_label, gt_label, pred_t, gt_t


# speedup vs baseline: 2.7768x; 1.0634x over previous
import math

import jax
import jax.numpy as jnp
from jax.experimental import pallas as pl
from jax.experimental.pallas import tpu as pltpu

TIMESTEPS = 1000
MAX_OUTLIERS = 128
N = 256
B = 8
SPS = 8
N_GOOD = 128
D = 256
DM = 256


def _ratio_table():
    epsilon = 0.008
    steps = jnp.linspace(0.0, TIMESTEPS, TIMESTEPS + 1, dtype=jnp.float32)
    f_t = jnp.cos((steps / TIMESTEPS + epsilon) / (1.0 + epsilon) * math.pi * 0.5) ** 2
    betas = jnp.clip(1.0 - f_t[1:] / f_t[:TIMESTEPS], 0.0, 0.999)
    alphas_cumprod = jnp.cumprod(1.0 - betas)
    tab = jnp.sqrt(1.0 - alphas_cumprod)
    tab = jnp.concatenate([tab, jnp.zeros((24,), jnp.float32)])
    return tab.reshape(8, 128)


def _one_sample(W, t, g, bd, pn_row, tab, temb):
    tf = t.astype(jnp.float32)

    r8 = jax.lax.broadcasted_iota(jnp.int32, (8, 128), 0)
    c8 = jax.lax.broadcasted_iota(jnp.int32, (8, 128), 1)
    ratio = jnp.sum(jnp.where(r8 * 128 + c8 == t, tab, 0.0))
    outlier = jnp.floor(MAX_OUTLIERS * ratio).astype(jnp.int32)
    datanum = N_GOOD + outlier

    io_i = jax.lax.broadcasted_iota(jnp.int32, (N, N), 0)
    io_j = jax.lax.broadcasted_iota(jnp.int32, (N, N), 1)

    x = jnp.concatenate([g, bd], axis=0)
    x = jnp.where(io_i < datanum, x, 1.0)

    h1 = jax.nn.relu(jnp.dot(x, W['Wd1']))
    h2 = jax.nn.relu(jnp.dot(h1, W['Wd2']))
    pooled = jnp.concatenate(
        [jnp.max(h2, axis=0, keepdims=True), jnp.mean(h2, axis=0, keepdims=True)],
        axis=1)
    pt_row = jax.nn.sigmoid(jnp.dot(pooled, W['wc']))

    xi = jnp.where(x >= 0.0, jnp.floor(x), jnp.ceil(x))
    feats = jnp.dot(xi, W['Win']) + temb
    q = jnp.dot(feats, W['Wq'])
    k = jnp.dot(feats, W['Wk'])
    v = jnp.dot(feats, W['Wv'])
    scores = jax.lax.dot_general(q, k, (((1,), (1,)), ((), ()))) * (1.0 / 16.0)
    m = jnp.max(scores, axis=1, keepdims=True)
    e = jnp.exp(scores - m)
    attn = e / jnp.sum(e, axis=1, keepdims=True)
    av = jnp.dot(attn, v)
    out = feats + jnp.dot(av, W['Wo'])
    logits = jnp.dot(jax.nn.relu(out), W['Wh'])

    keys_j = jnp.broadcast_to(pn_row, (N, N)) + jnp.where(io_j >= datanum, 1e6, 0.0)
    keys_i = keys_j.T
    before = (keys_j < keys_i) | ((keys_j == keys_i) & (io_j < io_i))
    rank = jnp.sum(before.astype(jnp.float32), axis=1, keepdims=True)
    Q = (rank == io_j.astype(jnp.float32)).astype(jnp.float32)

    icol = io_i[:, 0:1]
    lab = jnp.where(icol < N_GOOD, 0.0, jnp.where(icol < datanum, 1.0, -1.0))
    ch = jax.lax.broadcasted_iota(jnp.int32, (N, 128), 1)
    M = logits + jnp.where(ch == 2, lab, 0.0)

    final = jax.lax.dot_general(M, Q, (((0,), (0,)), ((), ())))

    gt = jnp.round(final[2:3, :]).astype(jnp.int32)
    li = jax.lax.broadcasted_iota(jnp.int32, (1, 256), 1)
    pt_ext = jnp.concatenate([pt_row, jnp.zeros((1, 128), jnp.float32)], axis=1)
    misc = jnp.where(li == 0, pt_ext,
                     jnp.where(li == 1, tf * (1.0 / TIMESTEPS), 0.0))
    return final[0:2, :], gt, misc


def _kern(t_sref, g_ref, b_ref, pn_ref, tab_ref,
          Win, Wt, Wq, Wk, Wv, Wo, Wh, Wd1, Wd2, wc,
          pl_ref, gt_ref, misc_ref):
    step = pl.program_id(0)
    W = dict(Win=Win[...], Wq=Wq[...], Wk=Wk[...], Wv=Wv[...], Wo=Wo[...],
             Wh=Wh[...], Wd1=Wd1[...], Wd2=Wd2[...], wc=wc[...])
    tab = tab_ref[...]

    lane = jax.lax.broadcasted_iota(jnp.int32, (SPS, 128), 1).astype(jnp.float32)
    tf_col = jnp.stack([t_sref[step * SPS + s].astype(jnp.float32)
                        for s in range(SPS)])[:, None]
    ang = tf_col * jnp.exp(-(math.log(10000.0) / 128.0) * lane)
    temb_all = jnp.dot(jnp.concatenate([jnp.sin(ang), jnp.cos(ang)], axis=1),
                       Wt[...])

    for s in range(SPS):
        t = t_sref[step * SPS + s]
        logits2, gt, misc = _one_sample(W, t, g_ref[s], b_ref[s], pn_ref[s],
                                        tab, temb_all[s:s + 1, :])
        pl_ref[s] = logits2
        gt_ref[s] = gt
        misc_ref[s] = misc


def kernel(good_tokens, bad_tokens, t, perm_noise,
           W_in, W_t, W_q, W_k, W_v, W_o, W_head, W_d1, W_d2, w_cls):
    tab = _ratio_table()
    pn_row = perm_noise.reshape(B, 1, N)
    Wh128 = jnp.pad(W_head, ((0, 0), (0, 126)))
    wc128 = jnp.pad(w_cls, ((0, 0), (0, 127)))

    full2d = lambda s: pl.BlockSpec(s, lambda i, *_: (0, 0))
    per_b = lambda s: pl.BlockSpec(s, lambda i, *_: (i, 0, 0))

    grid_spec = pltpu.PrefetchScalarGridSpec(
        num_scalar_prefetch=1,
        grid=(B // SPS,),
        in_specs=[
            per_b((SPS, N_GOOD, D)),
            per_b((SPS, N - N_GOOD, D)),
            per_b((SPS, 1, N)),
            full2d((8, 128)),
            full2d((D, DM)),
            full2d((DM, DM)),
            full2d((DM, DM)),
            full2d((DM, DM)),
            full2d((DM, DM)),
            full2d((DM, DM)),
            full2d((DM, 128)),
            full2d((D, 128)),
            full2d((128, 128)),
            full2d((256, 128)),
        ],
        out_specs=[
            per_b((SPS, 2, N)),
            per_b((SPS, 1, N)),
            per_b((SPS, 1, N)),
        ],
    )

    pred_label, gt3, misc = pl.pallas_call(
        _kern,
        grid_spec=grid_spec,
        out_shape=[
            jax.ShapeDtypeStruct((B, 2, N), jnp.float32),
            jax.ShapeDtypeStruct((B, 1, N), jnp.int32),
            jax.ShapeDtypeStruct((B, 1, N), jnp.float32),
        ],
    )(t, good_tokens, bad_tokens, pn_row, tab,
      W_in, W_t, W_q, W_k, W_v, W_o, Wh128, W_d1, W_d2, wc128)

    gt_label = gt3.reshape(B, N)
    pred_t = misc[:, 0, 0]
    gt_t = misc[:, 0, 1]
    return pred_label, gt_label, pred_t, gt_t
